# baseline (speedup 1.0000x reference)
import functools

import jax
import jax.numpy as jnp
from jax import lax
from jax.experimental import pallas as pl
from jax.experimental.pallas import tpu as pltpu
from jax.experimental.pallas import tpu_sc as plsc

_H = 128
_K = 8192
_L = 3
_B = 8192
_EPS = 1e-5
_PREC = lax.Precision.DEFAULT

_NC = 2
_NS = 16
_NW = _NC * _NS
_ROWS_PER_TILE = _B // _NW


def _ln(x, scale=None, bias=None):
    mu = jnp.mean(x, axis=-1, keepdims=True)
    var = jnp.mean((x - mu) ** 2, axis=-1, keepdims=True)
    y = (x - mu) / jnp.sqrt(var + _EPS)
    if scale is not None:
        y = y * scale + bias
    return y


def _gelu(x):
    return x * 0.5 * (1.0 + lax.erf(x * 0.7071067811865476))


_BBE = 512


def _encode_body(mw_ref, text_ref, vis_ref, wt_ref, bt_ref, st_ref, sbt_ref,
                 wv_ref, bv_ref, sv_ref, sbv_ref, out_ref):
    mw = mw_ref[...]
    e = jnp.exp(mw - jnp.max(mw))
    w = e / jnp.sum(e)
    t = jnp.dot(text_ref[...], wt_ref[...], precision=_PREC,
                preferred_element_type=jnp.float32) + bt_ref[...]
    enc_t = _gelu(_ln(t, st_ref[...], sbt_ref[...]))
    v = jnp.dot(vis_ref[...], wv_ref[...], precision=_PREC,
                preferred_element_type=jnp.float32) + bv_ref[...]
    enc_v = _gelu(_ln(v, sv_ref[...], sbv_ref[...]))
    fused = w[:, 0:1] * enc_t + w[:, 1:2] * enc_v
    nrm = jnp.sqrt(jnp.sum(fused * fused, axis=-1, keepdims=True))
    out_ref[...] = fused / jnp.maximum(nrm, 1e-12)


def _encode(mw, text, vis, wt, bt, st, sbt, wv, bv, sv, sbv):
    g = _B // _BBE
    full = lambda shape: pl.BlockSpec(shape, lambda i: (0,) * len(shape))
    return pl.pallas_call(
        _encode_body,
        grid=(g,),
        in_specs=[
            full((1, 2)),
            pl.BlockSpec((_BBE, text.shape[1]), lambda i: (i, 0)),
            pl.BlockSpec((_BBE, vis.shape[1]), lambda i: (i, 0)),
            full(wt.shape), full((1, _H)), full((1, _H)), full((1, _H)),
            full(wv.shape), full((1, _H)), full((1, _H)), full((1, _H)),
        ],
        out_specs=pl.BlockSpec((_BBE, _H), lambda i: (i, 0)),
        out_shape=jax.ShapeDtypeStruct((_B, _H), jnp.float32),
    )(mw, text, vis, wt, bt, st, sbt, wv, bv, sv, sbv)


_BBA = 2048
_BKA = 1024


def _argmin_body(nq, nk, *refs):
    fused_ref = refs[0]
    q_refs = refs[1:1 + nq]
    cb_ref = refs[1 + nq]
    out_ref = refs[2 + nq]
    best_ref, bidx_ref = refs[3 + nq:]
    k = pl.program_id(1)
    r = fused_ref[...]
    for qr in q_refs:
        r = r - qr[...]
    rr = jnp.sum(r * r, axis=-1, keepdims=True)
    cb = cb_ref[...]
    cbsq = jnp.sum(cb * cb, axis=-1)[None, :]
    rc = lax.dot_general(r.astype(jnp.bfloat16), cb.astype(jnp.bfloat16),
                         (((1,), (1,)), ((), ())),
                         preferred_element_type=jnp.float32)
    d = rr - 2.0 * rc + cbsq
    tmin = jnp.min(d, axis=-1, keepdims=True)
    cols = lax.broadcasted_iota(jnp.int32, d.shape, 1)
    targ = jnp.min(jnp.where(d == tmin, cols, _K), axis=-1, keepdims=True) + k * _BKA

    @pl.when(k == 0)
    def _():
        best_ref[...] = tmin
        bidx_ref[...] = targ

    @pl.when(k != 0)
    def _():
        upd = tmin < best_ref[...]
        bidx_ref[...] = jnp.where(upd, targ, bidx_ref[...])
        best_ref[...] = jnp.where(upd, tmin, best_ref[...])

    @pl.when(k == nk - 1)
    def _():
        out_ref[...] = bidx_ref[...]


def _argmin(fused, qs, cb):
    nq = len(qs)
    nb, nk = _B // _BBA, _K // _BKA
    blk = pl.BlockSpec((_BBA, _H), lambda i, k: (i, 0))
    return pl.pallas_call(
        functools.partial(_argmin_body, nq, nk),
        grid=(nb, nk),
        in_specs=[blk] * (1 + nq) + [pl.BlockSpec((_BKA, _H), lambda i, k: (k, 0))],
        out_specs=pl.BlockSpec((_BBA, 1), lambda i, k: (i, 0)),
        out_shape=jax.ShapeDtypeStruct((_B, 1), jnp.int32),
        scratch_shapes=[pltpu.VMEM((_BBA, 1), jnp.float32),
                        pltpu.VMEM((_BBA, 1), jnp.int32)],
    )(fused, *qs, cb)


def _sc_gather_body(cb_hbm, idx_hbm, q_hbm, cnt_hbm, idx_v, rows_v, cnt_v, sem):
    wid = lax.axis_index("s") * _NC + lax.axis_index("c")
    base = wid * _ROWS_PER_TILE
    pltpu.sync_copy(idx_hbm.at[pl.ds(base, _ROWS_PER_TILE)], idx_v)
    pltpu.async_copy(cb_hbm.at[idx_v], rows_v, sem).wait()
    pltpu.sync_copy(rows_v, q_hbm.at[pl.ds(base, _ROWS_PER_TILE)])

    zeros = jnp.zeros((16,), jnp.float32)

    def zbody(i, carry):
        cnt_v[pl.ds(i * 16, 16)] = zeros
        return carry

    lax.fori_loop(0, _K // 16, zbody, 0)

    ones = jnp.ones((16,), jnp.float32)

    def cbody(i, carry):
        iv = idx_v[pl.ds(i * 16, 16)]
        plsc.addupdate_scatter(cnt_v, [iv], ones)
        return carry

    lax.fori_loop(0, _ROWS_PER_TILE // 16, cbody, 0)
    pltpu.sync_copy(cnt_v, cnt_hbm.at[wid])


@functools.cache
def _sc_gather_kernel():
    return pl.kernel(
        _sc_gather_body,
        out_type=(jax.ShapeDtypeStruct((_B, _H), jnp.float32),
                  jax.ShapeDtypeStruct((_NW, _K), jnp.float32)),
        mesh=plsc.VectorSubcoreMesh(core_axis_name="c", subcore_axis_name="s",
                                    num_cores=_NC, num_subcores=_NS),
        compiler_params=pltpu.CompilerParams(needs_layout_passes=False),
        scratch_types=[pltpu.VMEM((_ROWS_PER_TILE,), jnp.int32),
                       pltpu.VMEM((_ROWS_PER_TILE, _H), jnp.float32),
                       pltpu.VMEM((_K,), jnp.float32),
                       pltpu.SemaphoreType.DMA],
    )


def _sc_gather(cb, idx):
    return _sc_gather_kernel()(cb, idx)


_BBF = 512


def _fuse_body(fused_ref, q0_ref, q1_ref, q2_ref, wf_ref, bf_ref, sf_ref,
               sbf_ref, cnt_ref, item_ref, qe_ref, recon_ref, resid_ref,
               bal_ref, acc_recon, acc_resid):
    i = pl.program_id(0)
    ng = pl.num_programs(0)
    f = fused_ref[...]
    q0 = q0_ref[...]
    q1 = q1_ref[...]
    q2 = q2_ref[...]
    quant = q0 + q1 + q2
    qst = f + (quant - f)
    qe = _ln(0.7 * qst + 0.3 * f)
    qe_ref[...] = qe
    comb = jnp.concatenate([f, qe], axis=-1)
    h = jnp.dot(comb, wf_ref[...], precision=_PREC,
                preferred_element_type=jnp.float32) + bf_ref[...]
    item_ref[...] = _gelu(_ln(h, sf_ref[...], sbf_ref[...]))

    recon_p = jnp.sum((qst - f) ** 2).reshape(1, 1)
    r1 = f - q0
    r2 = r1 - q1
    resid_p = (jnp.sum((q0 - f) ** 2) + jnp.sum((q1 - r1) ** 2)
               + jnp.sum((q2 - r2) ** 2)).reshape(1, 1)

    @pl.when(i == 0)
    def _():
        acc_recon[...] = recon_p
        acc_resid[...] = resid_p

    @pl.when(i != 0)
    def _():
        acc_recon[...] = acc_recon[...] + recon_p
        acc_resid[...] = acc_resid[...] + resid_p

    @pl.when(i == ng - 1)
    def _():
        denom = jnp.float32(_B * _H)
        recon_ref[...] = acc_recon[...] / denom
        resid_ref[...] = 1.25 * acc_resid[...] / denom
        cnt = cnt_ref[...]
        bl = jnp.zeros((1, 1), jnp.float32)
        logk = jnp.log(jnp.float32(_K))
        for l in range(_L):
            c = jnp.sum(cnt[l * _NW:(l + 1) * _NW, :], axis=0, keepdims=True)
            freq = c / (jnp.sum(c) + 1e-8)
            ent = -jnp.sum(freq * jnp.log(freq + 1e-8))
            bl = bl + (1.0 - ent / logk)
        bal_ref[...] = bl / _L


def _fuse(fused, q0, q1, q2, wf, bf, sf, sbf, counts):
    g = _B // _BBF
    blk = pl.BlockSpec((_BBF, _H), lambda i: (i, 0))
    full = lambda shape: pl.BlockSpec(shape, lambda i: (0,) * len(shape))
    one = pl.BlockSpec((1, 1), lambda i: (0, 0))
    return pl.pallas_call(
        _fuse_body,
        grid=(g,),
        in_specs=[blk, blk, blk, blk, full((2 * _H, _H)), full((1, _H)),
                  full((1, _H)), full((1, _H)), full((_L * _NW, _K))],
        out_specs=[blk, blk, one, one, one],
        out_shape=[jax.ShapeDtypeStruct((_B, _H), jnp.float32),
                   jax.ShapeDtypeStruct((_B, _H), jnp.float32),
                   jax.ShapeDtypeStruct((1, 1), jnp.float32),
                   jax.ShapeDtypeStruct((1, 1), jnp.float32),
                   jax.ShapeDtypeStruct((1, 1), jnp.float32)],
        scratch_shapes=[pltpu.VMEM((1, 1), jnp.float32),
                        pltpu.VMEM((1, 1), jnp.float32)],
    )(fused, q0, q1, q2, wf, bf, sf, sbf, counts)


def kernel(text_feat, vision_feat, W_text, b_text, ln_text_s, ln_text_b,
           W_vis, b_vis, ln_vis_s, ln_vis_b, modal_weight, codebooks,
           W_fuse, b_fuse, ln_fuse_s, ln_fuse_b):
    f32 = jnp.float32
    text = text_feat.reshape(-1, text_feat.shape[-1]).astype(f32)
    vis = vision_feat.reshape(-1, vision_feat.shape[-1]).astype(f32)
    row = lambda v: v.reshape(1, -1)

    fused = _encode(modal_weight.reshape(1, 2), text, vis,
                    W_text, row(b_text), row(ln_text_s), row(ln_text_b),
                    W_vis, row(b_vis), row(ln_vis_s), row(ln_vis_b))

    qs, idxs, cnts = [], [], []
    for l in range(_L):
        cb = codebooks[l]
        idx = _argmin(fused, qs, cb)
        q, cnt = _sc_gather(cb, idx.reshape(-1))
        idxs.append(idx)
        qs.append(q)
        cnts.append(cnt)

    counts = jnp.concatenate(cnts, axis=0)
    item, qe, recon, resid, bal = _fuse(fused, qs[0], qs[1], qs[2], W_fuse,
                                        row(b_fuse), row(ln_fuse_s),
                                        row(ln_fuse_b), counts)
    semantic_ids = jnp.concatenate(idxs, axis=1)
    return (item, qe, recon.reshape(()), resid.reshape(()), bal.reshape(()),
            semantic_ids)


# speedup vs baseline: 1.1045x; 1.1045x over previous
import functools

import jax
import jax.numpy as jnp
from jax import lax
from jax.experimental import pallas as pl
from jax.experimental.pallas import tpu as pltpu
from jax.experimental.pallas import tpu_sc as plsc

_H = 128
_K = 8192
_L = 3
_B = 8192
_EPS = 1e-5
_PREC = lax.Precision.DEFAULT

_NC = 2
_NS = 16
_NW = _NC * _NS
_ROWS_PER_TILE = _B // _NW


def _ln(x, scale=None, bias=None):
    mu = jnp.mean(x, axis=-1, keepdims=True)
    var = jnp.mean((x - mu) ** 2, axis=-1, keepdims=True)
    y = (x - mu) / jnp.sqrt(var + _EPS)
    if scale is not None:
        y = y * scale + bias
    return y


def _gelu(x):
    return x * 0.5 * (1.0 + lax.erf(x * 0.7071067811865476))


_BBE = 512


def _encode_body(mw_ref, text_ref, vis_ref, wt_ref, bt_ref, st_ref, sbt_ref,
                 wv_ref, bv_ref, sv_ref, sbv_ref, out_ref):
    mw = mw_ref[...]
    e = jnp.exp(mw - jnp.max(mw))
    w = e / jnp.sum(e)
    t = jnp.dot(text_ref[...], wt_ref[...], precision=_PREC,
                preferred_element_type=jnp.float32) + bt_ref[...]
    enc_t = _gelu(_ln(t, st_ref[...], sbt_ref[...]))
    v = jnp.dot(vis_ref[...], wv_ref[...], precision=_PREC,
                preferred_element_type=jnp.float32) + bv_ref[...]
    enc_v = _gelu(_ln(v, sv_ref[...], sbv_ref[...]))
    fused = w[:, 0:1] * enc_t + w[:, 1:2] * enc_v
    nrm = jnp.sqrt(jnp.sum(fused * fused, axis=-1, keepdims=True))
    out_ref[...] = fused / jnp.maximum(nrm, 1e-12)


def _encode(mw, text, vis, wt, bt, st, sbt, wv, bv, sv, sbv):
    g = _B // _BBE
    full = lambda shape: pl.BlockSpec(shape, lambda i: (0,) * len(shape))
    return pl.pallas_call(
        _encode_body,
        grid=(g,),
        in_specs=[
            full((1, 2)),
            pl.BlockSpec((_BBE, text.shape[1]), lambda i: (i, 0)),
            pl.BlockSpec((_BBE, vis.shape[1]), lambda i: (i, 0)),
            full(wt.shape), full((1, _H)), full((1, _H)), full((1, _H)),
            full(wv.shape), full((1, _H)), full((1, _H)), full((1, _H)),
        ],
        out_specs=pl.BlockSpec((_BBE, _H), lambda i: (i, 0)),
        out_shape=jax.ShapeDtypeStruct((_B, _H), jnp.float32),
    )(mw, text, vis, wt, bt, st, sbt, wv, bv, sv, sbv)


_BBA = 2048
_BKA = 1024


def _argmin_body(nq, nk, *refs):
    fused_ref = refs[0]
    q_refs = refs[1:1 + nq]
    cb_ref = refs[1 + nq]
    out_ref = refs[2 + nq]
    best_ref, bidx_ref = refs[3 + nq:]
    k = pl.program_id(1)
    r = fused_ref[...]
    for qr in q_refs:
        r = r - qr[...]
    cb = cb_ref[...]
    cbsq = jnp.sum(cb * cb, axis=-1)[None, :]
    rc2 = lax.dot_general((-2.0 * r).astype(jnp.bfloat16),
                          cb.astype(jnp.bfloat16),
                          (((1,), (1,)), ((), ())),
                          preferred_element_type=jnp.float32)
    d = rc2 + cbsq
    tmin = jnp.min(d, axis=-1, keepdims=True)
    cols = lax.broadcasted_iota(jnp.int32, d.shape, 1)
    targ = jnp.min(jnp.where(d == tmin, cols, _K), axis=-1, keepdims=True) + k * _BKA

    @pl.when(k == 0)
    def _():
        best_ref[...] = tmin
        bidx_ref[...] = targ

    @pl.when(k != 0)
    def _():
        upd = tmin < best_ref[...]
        bidx_ref[...] = jnp.where(upd, targ, bidx_ref[...])
        best_ref[...] = jnp.where(upd, tmin, best_ref[...])

    @pl.when(k == nk - 1)
    def _():
        out_ref[...] = bidx_ref[...]


def _argmin(fused, qs, cb):
    nq = len(qs)
    nb, nk = _B // _BBA, _K // _BKA
    blk = pl.BlockSpec((_BBA, _H), lambda i, k: (i, 0))
    return pl.pallas_call(
        functools.partial(_argmin_body, nq, nk),
        grid=(nb, nk),
        in_specs=[blk] * (1 + nq) + [pl.BlockSpec((_BKA, _H), lambda i, k: (k, 0))],
        out_specs=pl.BlockSpec((_BBA, 1), lambda i, k: (i, 0)),
        out_shape=jax.ShapeDtypeStruct((_B, 1), jnp.int32),
        scratch_shapes=[pltpu.VMEM((_BBA, 1), jnp.float32),
                        pltpu.VMEM((_BBA, 1), jnp.int32)],
    )(fused, *qs, cb)


def _sc_gather_body(cb_hbm, idx_hbm, zeros_hbm, q_hbm, cnt_hbm, idx_v, rows_v,
                    cnt_v, sem):
    wid = lax.axis_index("s") * _NC + lax.axis_index("c")
    base = wid * _ROWS_PER_TILE
    pltpu.sync_copy(idx_hbm.at[pl.ds(base, _ROWS_PER_TILE)], idx_v)
    pltpu.sync_copy(zeros_hbm, cnt_v)
    pltpu.async_copy(cb_hbm.at[idx_v], rows_v, sem).wait()
    pltpu.sync_copy(rows_v, q_hbm.at[pl.ds(base, _ROWS_PER_TILE)])

    ones = jnp.ones((16,), jnp.float32)

    def cbody(i, carry):
        iv = idx_v[pl.ds(i * 16, 16)]
        plsc.addupdate_scatter(cnt_v, [iv], ones)
        return carry

    lax.fori_loop(0, _ROWS_PER_TILE // 16, cbody, 0)
    pltpu.sync_copy(cnt_v, cnt_hbm.at[wid])


@functools.cache
def _sc_gather_kernel():
    return pl.kernel(
        _sc_gather_body,
        out_type=(jax.ShapeDtypeStruct((_B, _H), jnp.float32),
                  jax.ShapeDtypeStruct((_NW, _K), jnp.float32)),
        mesh=plsc.VectorSubcoreMesh(core_axis_name="c", subcore_axis_name="s",
                                    num_cores=_NC, num_subcores=_NS),
        compiler_params=pltpu.CompilerParams(needs_layout_passes=False),
        scratch_types=[pltpu.VMEM((_ROWS_PER_TILE,), jnp.int32),
                       pltpu.VMEM((_ROWS_PER_TILE, _H), jnp.float32),
                       pltpu.VMEM((_K,), jnp.float32),
                       pltpu.SemaphoreType.DMA],
    )


def _sc_gather(cb, idx):
    return _sc_gather_kernel()(cb, idx, jnp.zeros((_K,), jnp.float32))


_BBF = 512


def _fuse_body(fused_ref, q0_ref, q1_ref, q2_ref, wf_ref, bf_ref, sf_ref,
               sbf_ref, cnt_ref, item_ref, qe_ref, recon_ref, resid_ref,
               bal_ref, acc_recon, acc_resid):
    i = pl.program_id(0)
    ng = pl.num_programs(0)
    f = fused_ref[...]
    q0 = q0_ref[...]
    q1 = q1_ref[...]
    q2 = q2_ref[...]
    quant = q0 + q1 + q2
    qst = f + (quant - f)
    qe = _ln(0.7 * qst + 0.3 * f)
    qe_ref[...] = qe
    comb = jnp.concatenate([f, qe], axis=-1)
    h = jnp.dot(comb, wf_ref[...], precision=_PREC,
                preferred_element_type=jnp.float32) + bf_ref[...]
    item_ref[...] = _gelu(_ln(h, sf_ref[...], sbf_ref[...]))

    recon_p = jnp.sum((qst - f) ** 2).reshape(1, 1)
    r1 = f - q0
    r2 = r1 - q1
    resid_p = (jnp.sum((q0 - f) ** 2) + jnp.sum((q1 - r1) ** 2)
               + jnp.sum((q2 - r2) ** 2)).reshape(1, 1)

    @pl.when(i == 0)
    def _():
        acc_recon[...] = recon_p
        acc_resid[...] = resid_p

    @pl.when(i != 0)
    def _():
        acc_recon[...] = acc_recon[...] + recon_p
        acc_resid[...] = acc_resid[...] + resid_p

    @pl.when(i == ng - 1)
    def _():
        denom = jnp.float32(_B * _H)
        recon_ref[...] = acc_recon[...] / denom
        resid_ref[...] = 1.25 * acc_resid[...] / denom
        cnt = cnt_ref[...]
        bl = jnp.zeros((1, 1), jnp.float32)
        logk = jnp.log(jnp.float32(_K))
        for l in range(_L):
            c = jnp.sum(cnt[l * _NW:(l + 1) * _NW, :], axis=0, keepdims=True)
            freq = c / (jnp.sum(c) + 1e-8)
            ent = -jnp.sum(freq * jnp.log(freq + 1e-8))
            bl = bl + (1.0 - ent / logk)
        bal_ref[...] = bl / _L


def _fuse(fused, q0, q1, q2, wf, bf, sf, sbf, counts):
    g = _B // _BBF
    blk = pl.BlockSpec((_BBF, _H), lambda i: (i, 0))
    full = lambda shape: pl.BlockSpec(shape, lambda i: (0,) * len(shape))
    one = pl.BlockSpec((1, 1), lambda i: (0, 0))
    return pl.pallas_call(
        _fuse_body,
        grid=(g,),
        in_specs=[blk, blk, blk, blk, full((2 * _H, _H)), full((1, _H)),
                  full((1, _H)), full((1, _H)), full((_L * _NW, _K))],
        out_specs=[blk, blk, one, one, one],
        out_shape=[jax.ShapeDtypeStruct((_B, _H), jnp.float32),
                   jax.ShapeDtypeStruct((_B, _H), jnp.float32),
                   jax.ShapeDtypeStruct((1, 1), jnp.float32),
                   jax.ShapeDtypeStruct((1, 1), jnp.float32),
                   jax.ShapeDtypeStruct((1, 1), jnp.float32)],
        scratch_shapes=[pltpu.VMEM((1, 1), jnp.float32),
                        pltpu.VMEM((1, 1), jnp.float32)],
    )(fused, q0, q1, q2, wf, bf, sf, sbf, counts)


def kernel(text_feat, vision_feat, W_text, b_text, ln_text_s, ln_text_b,
           W_vis, b_vis, ln_vis_s, ln_vis_b, modal_weight, codebooks,
           W_fuse, b_fuse, ln_fuse_s, ln_fuse_b):
    f32 = jnp.float32
    text = text_feat.reshape(-1, text_feat.shape[-1]).astype(f32)
    vis = vision_feat.reshape(-1, vision_feat.shape[-1]).astype(f32)
    row = lambda v: v.reshape(1, -1)

    fused = _encode(modal_weight.reshape(1, 2), text, vis,
                    W_text, row(b_text), row(ln_text_s), row(ln_text_b),
                    W_vis, row(b_vis), row(ln_vis_s), row(ln_vis_b))

    qs, idxs, cnts = [], [], []
    for l in range(_L):
        cb = codebooks[l]
        idx = _argmin(fused, qs, cb)
        q, cnt = _sc_gather(cb, idx.reshape(-1))
        idxs.append(idx)
        qs.append(q)
        cnts.append(cnt)

    counts = jnp.concatenate(cnts, axis=0)
    item, qe, recon, resid, bal = _fuse(fused, qs[0], qs[1], qs[2], W_fuse,
                                        row(b_fuse), row(ln_fuse_s),
                                        row(ln_fuse_b), counts)
    semantic_ids = jnp.concatenate(idxs, axis=1)
    return (item, qe, recon.reshape(()), resid.reshape(()), bal.reshape(()),
            semantic_ids)
